# baseline (speedup 1.0000x reference)
import functools

import jax
import jax.numpy as jnp
from jax import lax
from jax.experimental import pallas as pl
from jax.experimental.pallas import tpu as pltpu
from jax.experimental.pallas import tpu_sc as plsc

BATCH = 16384
D = 128
GATHER_WINDOW = 128
CHUNK_SIZES = (8192, 8192)
MLP_BLOCK = 8192


NUM_CORES = 2
NUM_SUBCORES = 16
NUM_WORKERS = NUM_CORES * NUM_SUBCORES


def _gather_body(wins_per_worker, uf_hbm, if_hbm, ui_hbm, ii_hbm,
                 ue_hbm, ie_hbm, idx_u, idx_i, ru, ri, gsem, wsem):
    w = GATHER_WINDOW
    rows_per_worker = wins_per_worker * w
    wid = lax.axis_index("s") * NUM_CORES + lax.axis_index("c")
    base_win = wid * wins_per_worker
    pltpu.sync_copy(ui_hbm.at[pl.ds(base_win, wins_per_worker)], idx_u)
    pltpu.sync_copy(ii_hbm.at[pl.ds(base_win, wins_per_worker)], idx_i)
    gu, gi = [], []
    for j in range(wins_per_worker):
        gu.append(pltpu.make_async_copy(
            uf_hbm.at[idx_u.at[j]], ru.at[pl.ds(j * w, w)], gsem.at[2 * j]))
        gi.append(pltpu.make_async_copy(
            if_hbm.at[idx_i.at[j]], ri.at[pl.ds(j * w, w)],
            gsem.at[2 * j + 1]))
    for c in gu + gi:
        c.start()
    base_row = wid * rows_per_worker
    for c in gu:
        c.wait()
    wu = pltpu.make_async_copy(
        ru, ue_hbm.at[pl.ds(base_row, rows_per_worker)], wsem.at[0])
    wu.start()
    for c in gi:
        c.wait()
    wi = pltpu.make_async_copy(
        ri, ie_hbm.at[pl.ds(base_row, rows_per_worker)], wsem.at[1])
    wi.start()
    wu.wait()
    wi.wait()


def _sc_gather(user_factors, item_factors, user2d, item2d):
    n = user2d.shape[0] * GATHER_WINDOW
    wins_per_worker = user2d.shape[0] // NUM_WORKERS
    rows_per_worker = wins_per_worker * GATHER_WINDOW
    mesh = plsc.VectorSubcoreMesh(core_axis_name="c", subcore_axis_name="s")
    f = pl.kernel(
        functools.partial(_gather_body, wins_per_worker),
        out_type=(
            jax.ShapeDtypeStruct((n, D), jnp.float32),
            jax.ShapeDtypeStruct((n, D), jnp.float32),
        ),
        mesh=mesh,
        scratch_types=[
            pltpu.VMEM((wins_per_worker, GATHER_WINDOW), jnp.int32),
            pltpu.VMEM((wins_per_worker, GATHER_WINDOW), jnp.int32),
            pltpu.VMEM((rows_per_worker, D), jnp.float32),
            pltpu.VMEM((rows_per_worker, D), jnp.float32),
            pltpu.SemaphoreType.DMA((2 * wins_per_worker,)),
            pltpu.SemaphoreType.DMA((2,)),
        ],
    )
    return f(user_factors, item_factors, user2d, item2d)


def _mlp_body(ue_ref, ie_ref, w1u_ref, w1i_ref, b1_ref, w2_ref, b2_ref,
              w3_ref, b3_ref, o_ref):
    dn = (((1,), (1,)), ((), ()))
    h = lax.dot_general(ue_ref[...], w1u_ref[...], dn,
                        preferred_element_type=jnp.float32)
    h = h + lax.dot_general(ie_ref[...], w1i_ref[...], dn,
                            preferred_element_type=jnp.float32)
    h = jnp.maximum(h + b1_ref[...][None, :], 0.0)
    h = lax.dot_general(h, w2_ref[...], dn, preferred_element_type=jnp.float32)
    h = jnp.maximum(h + b2_ref[...][None, :], 0.0)
    h = lax.dot_general(w3_ref[...], h, dn, preferred_element_type=jnp.float32)
    o_ref[...] = jax.nn.sigmoid(h + b3_ref[...][:, None])


def _tc_mlp(ue, ie, W1, b1, W2, b2, W3, b3):
    n = ue.shape[0]
    blk = min(MLP_BLOCK, n)
    full = lambda shape: pl.BlockSpec(shape, lambda i: tuple(0 for _ in shape))
    return pl.pallas_call(
        _mlp_body,
        grid=(n // blk,),
        in_specs=[
            pl.BlockSpec((blk, D), lambda i: (i, 0)),
            pl.BlockSpec((blk, D), lambda i: (i, 0)),
            pl.BlockSpec((D, D), lambda i: (0, 0)),
        pl.BlockSpec((D, D), lambda i: (0, 1)),
        full(b1.shape),
            full(W2.shape), full(b2.shape), full(W3.shape), full(b3.shape),
        ],
        out_specs=pl.BlockSpec((2, blk), lambda i: (0, i)),
        out_shape=jax.ShapeDtypeStruct((2, n), jnp.float32),
    )(ue, ie, W1, W1, b1, W2, b2, W3, b3)


def kernel(user, item, user_factors, item_factors, W1, b1, W2, b2, W3, b3):
    user2d = user.astype(jnp.int32).reshape(
        BATCH // GATHER_WINDOW, GATHER_WINDOW)
    item2d = item.astype(jnp.int32).reshape(
        BATCH // GATHER_WINDOW, GATHER_WINDOW)
    embs = []
    off = 0
    for c in CHUNK_SIZES:
        nw = c // GATHER_WINDOW
        embs.append(_sc_gather(
            user_factors, item_factors,
            lax.slice_in_dim(user2d, off, off + nw),
            lax.slice_in_dim(item2d, off, off + nw)))
        off += nw
    outs = [
        _tc_mlp(ue, ie, W1, b1, W2, b2, W3, b3) for ue, ie in embs
    ]
    return jnp.concatenate(outs, axis=1).T


# speedup vs baseline: 1.0246x; 1.0246x over previous
import functools

import jax
import jax.numpy as jnp
from jax import lax
from jax.experimental import pallas as pl
from jax.experimental.pallas import tpu as pltpu
from jax.experimental.pallas import tpu_sc as plsc

BATCH = 16384
D = 128
GATHER_WINDOW = 128
CHUNK_SIZES = (8192, 8192)
MLP_BLOCK = 2048


NUM_CORES = 2
NUM_SUBCORES = 16
NUM_WORKERS = NUM_CORES * NUM_SUBCORES


def _gather_body(wins_per_worker, uf_hbm, if_hbm, ui_hbm, ii_hbm,
                 ue_hbm, ie_hbm, idx_u, idx_i, ru, ri, gsem, wsem):
    w = GATHER_WINDOW
    rows_per_worker = wins_per_worker * w
    wid = lax.axis_index("s") * NUM_CORES + lax.axis_index("c")
    base_win = wid * wins_per_worker
    pltpu.sync_copy(ui_hbm.at[pl.ds(base_win, wins_per_worker)], idx_u)
    pltpu.sync_copy(ii_hbm.at[pl.ds(base_win, wins_per_worker)], idx_i)
    gu, gi = [], []
    for j in range(wins_per_worker):
        gu.append(pltpu.make_async_copy(
            uf_hbm.at[idx_u.at[j]], ru.at[pl.ds(j * w, w)], gsem.at[2 * j]))
        gi.append(pltpu.make_async_copy(
            if_hbm.at[idx_i.at[j]], ri.at[pl.ds(j * w, w)],
            gsem.at[2 * j + 1]))
    for c in gu + gi:
        c.start()
    base_row = wid * rows_per_worker
    for c in gu:
        c.wait()
    wu = pltpu.make_async_copy(
        ru, ue_hbm.at[pl.ds(base_row, rows_per_worker)], wsem.at[0])
    wu.start()
    for c in gi:
        c.wait()
    wi = pltpu.make_async_copy(
        ri, ie_hbm.at[pl.ds(base_row, rows_per_worker)], wsem.at[1])
    wi.start()
    wu.wait()
    wi.wait()


def _sc_gather(user_factors, item_factors, user2d, item2d):
    n = user2d.shape[0] * GATHER_WINDOW
    wins_per_worker = user2d.shape[0] // NUM_WORKERS
    rows_per_worker = wins_per_worker * GATHER_WINDOW
    mesh = plsc.VectorSubcoreMesh(core_axis_name="c", subcore_axis_name="s")
    f = pl.kernel(
        functools.partial(_gather_body, wins_per_worker),
        out_type=(
            jax.ShapeDtypeStruct((n, D), jnp.float32),
            jax.ShapeDtypeStruct((n, D), jnp.float32),
        ),
        mesh=mesh,
        scratch_types=[
            pltpu.VMEM((wins_per_worker, GATHER_WINDOW), jnp.int32),
            pltpu.VMEM((wins_per_worker, GATHER_WINDOW), jnp.int32),
            pltpu.VMEM((rows_per_worker, D), jnp.float32),
            pltpu.VMEM((rows_per_worker, D), jnp.float32),
            pltpu.SemaphoreType.DMA((2 * wins_per_worker,)),
            pltpu.SemaphoreType.DMA((2,)),
        ],
    )
    return f(user_factors, item_factors, user2d, item2d)


def _mlp_body(ue_ref, ie_ref, w1u_ref, w1i_ref, b1_ref, w2_ref, b2_ref,
              w3_ref, b3_ref, o_ref):
    dn = (((1,), (1,)), ((), ()))
    h = lax.dot_general(ue_ref[...], w1u_ref[...], dn,
                        preferred_element_type=jnp.float32)
    h = h + lax.dot_general(ie_ref[...], w1i_ref[...], dn,
                            preferred_element_type=jnp.float32)
    h = jnp.maximum(h + b1_ref[...][None, :], 0.0)
    h = lax.dot_general(h, w2_ref[...], dn, preferred_element_type=jnp.float32)
    h = jnp.maximum(h + b2_ref[...][None, :], 0.0)
    h = lax.dot_general(w3_ref[...], h, dn, preferred_element_type=jnp.float32)
    o_ref[...] = jax.nn.sigmoid(h + b3_ref[...][:, None])


def _tc_mlp(ue, ie, W1, b1, W2, b2, W3, b3):
    n = ue.shape[0]
    blk = min(MLP_BLOCK, n)
    full = lambda shape: pl.BlockSpec(shape, lambda i: tuple(0 for _ in shape))
    return pl.pallas_call(
        _mlp_body,
        grid=(n // blk,),
        in_specs=[
            pl.BlockSpec((blk, D), lambda i: (i, 0)),
            pl.BlockSpec((blk, D), lambda i: (i, 0)),
            pl.BlockSpec((D, D), lambda i: (0, 0)),
        pl.BlockSpec((D, D), lambda i: (0, 1)),
        full(b1.shape),
            full(W2.shape), full(b2.shape), full(W3.shape), full(b3.shape),
        ],
        out_specs=pl.BlockSpec((2, blk), lambda i: (0, i)),
        out_shape=jax.ShapeDtypeStruct((2, n), jnp.float32),
    )(ue, ie, W1, W1, b1, W2, b2, W3, b3)


def kernel(user, item, user_factors, item_factors, W1, b1, W2, b2, W3, b3):
    user2d = user.astype(jnp.int32).reshape(
        BATCH // GATHER_WINDOW, GATHER_WINDOW)
    item2d = item.astype(jnp.int32).reshape(
        BATCH // GATHER_WINDOW, GATHER_WINDOW)
    embs = []
    off = 0
    for c in CHUNK_SIZES:
        nw = c // GATHER_WINDOW
        embs.append(_sc_gather(
            user_factors, item_factors,
            lax.slice_in_dim(user2d, off, off + nw),
            lax.slice_in_dim(item2d, off, off + nw)))
        off += nw
    outs = [
        _tc_mlp(ue, ie, W1, b1, W2, b2, W3, b3) for ue, ie in embs
    ]
    return jnp.concatenate(outs, axis=1).T
